# baseline (speedup 1.0000x reference)
import functools

import jax
import jax.numpy as jnp
from jax import lax
from jax.experimental import pallas as pl
from jax.experimental.pallas import tpu as pltpu
from jax.experimental.pallas import tpu_sc as plsc

_N = 10000
_D = 128
_NC = 2
_NS = 16
_NW = _NC * _NS
_L = 16
_CH = 128
_NBUF = 2
_C0 = 160
_C1 = 0
_NCH = _NS * (_C0 + _C1)
_NPAD = 10240
_RPT = _NPAD // _NS


def _zero_1d(ref, n):
    z = jnp.zeros((_L,), jnp.float32)

    def bd(i, c):
        ref[pl.ds(i * _L, _L)] = z
        return c

    lax.fori_loop(0, n // _L, bd, 0)


def _zero_2d(ref, nrows, ncols):
    z = jnp.zeros((_L,), jnp.float32)
    cpr = ncols // _L

    def bd(i, c):
        ref[i // cpr, pl.ds((i % cpr) * _L, _L)] = z
        return c

    lax.fori_loop(0, nrows * cpr, bd, 0)


def _make_sc_agg(with_deg):
    mesh = plsc.VectorSubcoreMesh(core_axis_name="c", subcore_axis_name="s")
    qc = 16
    out_type = [jax.ShapeDtypeStruct((_NC, _NPAD, _D), jnp.float32)]
    scratch = [
        pltpu.VMEM((qc, _CH), jnp.int32),
        pltpu.VMEM((qc, _CH), jnp.int32),
        pltpu.VMEM((_NBUF, _CH, _D), jnp.float32),
        pltpu.VMEM_SHARED((_NPAD, _D), jnp.float32),
        pltpu.SemaphoreType.DMA,
        pltpu.SemaphoreType.DMA,
    ]
    if with_deg:
        out_type.append(jax.ShapeDtypeStruct((_NW, _NPAD), jnp.float32))
        scratch.append(pltpu.VMEM((_NPAD,), jnp.float32))

    def body(table_hbm, srcp_hbm, dstp_hbm, acc_hbm, *rest):
        if with_deg:
            deg_hbm, src_v, dst_v, rows_v, acc_sh, gsem, ssem, hist_v = rest
        else:
            src_v, dst_v, rows_v, acc_sh, gsem, ssem = rest
        cid = lax.axis_index("c")
        sid = lax.axis_index("s")
        wid = sid * _NC + cid

        _zero_2d(rows_v.at[0], _CH, _D)
        for k in range(_RPT // _CH):
            pltpu.sync_copy(rows_v.at[0],
                            acc_sh.at[pl.ds(sid * _RPT + k * _CH, _CH)])
        if with_deg:
            _zero_1d(hist_v, _NPAD)
        plsc.subcore_barrier()

        ones = jnp.ones((_L,), jnp.float32)

        def start_gather(x, b):
            pltpu.async_copy(table_hbm.at[src_v.at[x]], rows_v.at[b], gsem)

        def wait_gather(b):
            pltpu.make_async_copy(table_hbm.at[src_v.at[0]], rows_v.at[b],
                                  gsem).wait()

        def start_scatter(x, b):
            pltpu.async_copy(rows_v.at[b], acc_sh.at[dst_v.at[x]], ssem,
                             add=True)

        def wait_scatter(b):
            pltpu.make_async_copy(rows_v.at[b], acc_sh.at[dst_v.at[0]],
                                  ssem).wait()

        def hist(x):
            if with_deg:
                for j in range(_CH // _L):
                    plsc.addupdate_scatter(
                        hist_v, [dst_v[x, pl.ds(j * _L, _L)]], ones)

        def pipe_step(x, bcur, prefetch):
            wait_gather(bcur)
            wait_scatter(1 - bcur)
            if prefetch:
                start_gather(x + 1, 1 - bcur)
            start_scatter(x, bcur)
            hist(x)

        chunk0 = jnp.where(cid == 0, sid * _C0, _NS * _C0 + sid * _C1)
        nb = jnp.where(cid == 0, _C0 // qc, _C1 // qc)
        active = (cid == 0) if _C1 == 0 else (cid >= 0)

        def run_batch(q, first):
            off = chunk0 + q * qc
            pltpu.sync_copy(srcp_hbm.at[pl.ds(off, qc)], src_v)
            pltpu.sync_copy(dstp_hbm.at[pl.ds(off, qc)], dst_v)
            start_gather(0, 0)
            if first:
                wait_gather(0)
                start_gather(1, 1)
                start_scatter(0, 0)
                hist(0)
            else:
                pipe_step(0, 0, True)

            def pair(p, c):
                pipe_step(1 + 2 * p, 1, True)
                pipe_step(2 + 2 * p, 0, True)
                return c

            lax.fori_loop(0, (qc - 2) // 2, pair, 0)
            pipe_step(qc - 1, 1, False)

        @pl.when(active)
        def _():
            run_batch(0, True)
            lax.fori_loop(1, nb, lambda q, c: (run_batch(q, False), c)[1], 0)
            wait_scatter(1)

        plsc.subcore_barrier()

        r0 = sid * _RPT
        pltpu.sync_copy(acc_sh.at[pl.ds(r0, _RPT)],
                        acc_hbm.at[cid].at[pl.ds(r0, _RPT)])
        if with_deg:
            pltpu.sync_copy(hist_v, deg_hbm.at[wid])

    return pl.kernel(
        body, out_type=tuple(out_type), mesh=mesh, scratch_types=scratch,
        compiler_params=pltpu.CompilerParams(needs_layout_passes=False))


_sc_agg_deg = _make_sc_agg(True)
_sc_agg = _make_sc_agg(False)


def _tc_layer(p, degp, w, b, relu):
    br = 2048

    def body(p_ref, degp_ref, w_ref, b_ref, o_ref):
        deg = jnp.maximum(jnp.sum(degp_ref[...], axis=0), 1.0)
        s = p_ref[0] + p_ref[1]
        hn = s * (1.0 / deg)[:, None]
        y = jnp.dot(hn, w_ref[...], preferred_element_type=jnp.float32)
        y = y + b_ref[...]
        o_ref[...] = jnp.maximum(y, 0.0) if relu else y

    return pl.pallas_call(
        body,
        grid=(_NPAD // br,),
        in_specs=[
            pl.BlockSpec((_NC, br, _D), lambda i: (0, i, 0)),
            pl.BlockSpec((_NW, br), lambda i: (0, i)),
            pl.BlockSpec((_D, _D), lambda i: (0, 0)),
            pl.BlockSpec((1, _D), lambda i: (0, 0)),
        ],
        out_specs=pl.BlockSpec((br, _D), lambda i: (i, 0)),
        out_shape=jax.ShapeDtypeStruct((_NPAD, _D), jnp.float32),
    )(p, degp, w, b.reshape(1, _D))


def kernel(feature, edge_index, W1, b1, W2, b2, W3, b3):
    del W1, b1
    e = edge_index.shape[1]
    e_pad = _NCH * _CH
    src = jnp.concatenate(
        [edge_index[0], jnp.zeros((e_pad - e,), jnp.int32)])
    dst = jnp.concatenate(
        [edge_index[1],
         _N + jnp.arange(e_pad - e, dtype=jnp.int32) % (_NPAD - _N)])
    srcp = src.reshape(_NCH, _CH)
    dstp = dst.reshape(_NCH, _CH)

    accp1, degp = _sc_agg_deg(feature, srcp, dstp)
    h2 = _tc_layer(accp1, degp, W2, b2, True)
    (accp2,) = _sc_agg(h2, srcp, dstp)
    return _tc_layer(accp2, degp, W3, b3, False)[:_N]


# speedup vs baseline: 1.3558x; 1.3558x over previous
import functools

import jax
import jax.numpy as jnp
from jax import lax
from jax.experimental import pallas as pl
from jax.experimental.pallas import tpu as pltpu
from jax.experimental.pallas import tpu_sc as plsc

_N = 10000
_D = 128
_NC = 2
_NS = 16
_NW = _NC * _NS
_L = 16
_CH = 128
_NBUF = 2
_C0 = 128
_C1 = 32
_NCH = _NS * (_C0 + _C1)
_NPAD = 10240
_RPT = _NPAD // _NS


def _zero_1d(ref, n):
    z = jnp.zeros((_L,), jnp.float32)

    def bd(i, c):
        ref[pl.ds(i * _L, _L)] = z
        return c

    lax.fori_loop(0, n // _L, bd, 0)


def _zero_2d(ref, nrows, ncols):
    z = jnp.zeros((_L,), jnp.float32)
    cpr = ncols // _L

    def bd(i, c):
        ref[i // cpr, pl.ds((i % cpr) * _L, _L)] = z
        return c

    lax.fori_loop(0, nrows * cpr, bd, 0)


def _make_sc_agg(with_deg):
    mesh = plsc.VectorSubcoreMesh(core_axis_name="c", subcore_axis_name="s")
    qc = 8
    out_type = [jax.ShapeDtypeStruct((_NC, _NPAD, _D), jnp.float32)]
    scratch = [
        pltpu.VMEM((2, qc, _CH), jnp.int32),
        pltpu.VMEM((2, qc, _CH), jnp.int32),
        pltpu.VMEM((_NBUF, _CH, _D), jnp.float32),
        pltpu.VMEM_SHARED((_NPAD, _D), jnp.float32),
        pltpu.SemaphoreType.DMA,
        pltpu.SemaphoreType.DMA,
        pltpu.SemaphoreType.DMA,
    ]
    if with_deg:
        out_type.append(jax.ShapeDtypeStruct((_NW, _NPAD), jnp.float32))
        scratch.append(pltpu.VMEM((_NPAD,), jnp.float32))

    def body(table_hbm, srcp_hbm, dstp_hbm, acc_hbm, *rest):
        if with_deg:
            (deg_hbm, src_v, dst_v, rows_v, acc_sh, gsem, ssem, isem,
             hist_v) = rest
        else:
            src_v, dst_v, rows_v, acc_sh, gsem, ssem, isem = rest
        cid = lax.axis_index("c")
        sid = lax.axis_index("s")
        wid = sid * _NC + cid

        _zero_2d(rows_v.at[0], _CH, _D)
        for k in range(_RPT // _CH):
            pltpu.sync_copy(rows_v.at[0],
                            acc_sh.at[pl.ds(sid * _RPT + k * _CH, _CH)])
        if with_deg:
            _zero_1d(hist_v, _NPAD)
        plsc.subcore_barrier()

        ones = jnp.ones((_L,), jnp.float32)

        def wait_gather(b):
            pltpu.make_async_copy(table_hbm.at[src_v.at[0, 0]],
                                  rows_v.at[b], gsem).wait()

        def wait_scatter(b):
            pltpu.make_async_copy(rows_v.at[b], acc_sh.at[dst_v.at[0, 0]],
                                  ssem).wait()

        def run_core(nchk, base):
            nb = nchk // qc

            def idx_load(q, par):
                off = base + q * qc
                pltpu.async_copy(srcp_hbm.at[pl.ds(off, qc)],
                                 src_v.at[par], isem)
                pltpu.async_copy(dstp_hbm.at[pl.ds(off, qc)],
                                 dst_v.at[par], isem)

            def idx_wait():
                for _ in range(2):
                    pltpu.make_async_copy(srcp_hbm.at[pl.ds(0, qc)],
                                          src_v.at[0], isem).wait()

            def start_gather(x, b):
                par = (x // qc) % 2
                pltpu.async_copy(table_hbm.at[src_v.at[par, x % qc]],
                                 rows_v.at[b], gsem)

            def start_scatter(x, b):
                par = (x // qc) % 2
                pltpu.async_copy(rows_v.at[b],
                                 acc_sh.at[dst_v.at[par, x % qc]], ssem,
                                 add=True)

            def hist_x(x):
                if with_deg:
                    par = (x // qc) % 2
                    s = x % qc
                    for j in range(_CH // _L):
                        plsc.addupdate_scatter(
                            hist_v, [dst_v[par, s, pl.ds(j * _L, _L)]],
                            ones)

            idx_load(0, 0)
            idx_wait()
            start_gather(0, 0)

            def substep(x, b):
                q = x // qc
                slot = x % qc
                wait_gather(b)

                @pl.when(x > 0)
                def _ws():
                    wait_scatter(1 - b)

                @pl.when(jnp.logical_and(slot == 2, q + 1 < nb))
                def _pf():
                    idx_load(q + 1, (q + 1) % 2)

                @pl.when(x + 1 < nchk)
                def _g():
                    @pl.when(slot == qc - 1)
                    def _wi():
                        idx_wait()

                    start_gather(x + 1, 1 - b)

                start_scatter(x, b)
                hist_x(x)

            def pair(p, c):
                substep(2 * p, 0)
                substep(2 * p + 1, 1)
                return c

            lax.fori_loop(0, nchk // 2, pair, 0)
            wait_scatter(1)

        @pl.when(cid == 0)
        def _core0():
            run_core(_C0, sid * _C0)

        if _C1 > 0:
            @pl.when(cid == 1)
            def _core1():
                run_core(_C1, _NS * _C0 + sid * _C1)

        plsc.subcore_barrier()

        r0 = sid * _RPT
        pltpu.sync_copy(acc_sh.at[pl.ds(r0, _RPT)],
                        acc_hbm.at[cid].at[pl.ds(r0, _RPT)])
        if with_deg:
            pltpu.sync_copy(hist_v, deg_hbm.at[wid])

    return pl.kernel(
        body, out_type=tuple(out_type), mesh=mesh, scratch_types=scratch,
        compiler_params=pltpu.CompilerParams(needs_layout_passes=False))


_sc_agg_deg = _make_sc_agg(True)
_sc_agg = _make_sc_agg(False)


def _tc_layer(p, degp, w, b, relu):
    br = 2048

    def body(p_ref, degp_ref, w_ref, b_ref, o_ref):
        deg = jnp.maximum(jnp.sum(degp_ref[...], axis=0), 1.0)
        s = p_ref[0] + p_ref[1]
        hn = s * (1.0 / deg)[:, None]
        y = jnp.dot(hn, w_ref[...], preferred_element_type=jnp.float32)
        y = y + b_ref[...]
        o_ref[...] = jnp.maximum(y, 0.0) if relu else y

    return pl.pallas_call(
        body,
        grid=(_NPAD // br,),
        in_specs=[
            pl.BlockSpec((_NC, br, _D), lambda i: (0, i, 0)),
            pl.BlockSpec((_NW, br), lambda i: (0, i)),
            pl.BlockSpec((_D, _D), lambda i: (0, 0)),
            pl.BlockSpec((1, _D), lambda i: (0, 0)),
        ],
        out_specs=pl.BlockSpec((br, _D), lambda i: (i, 0)),
        out_shape=jax.ShapeDtypeStruct((_NPAD, _D), jnp.float32),
    )(p, degp, w, b.reshape(1, _D))


def kernel(feature, edge_index, W1, b1, W2, b2, W3, b3):
    del W1, b1
    e = edge_index.shape[1]
    e_pad = _NCH * _CH
    src = jnp.concatenate(
        [edge_index[0], jnp.zeros((e_pad - e,), jnp.int32)])
    dst = jnp.concatenate(
        [edge_index[1],
         _N + jnp.arange(e_pad - e, dtype=jnp.int32) % (_NPAD - _N)])
    srcp = src.reshape(_NCH, _CH)
    dstp = dst.reshape(_NCH, _CH)

    accp1, degp = _sc_agg_deg(feature, srcp, dstp)
    h2 = _tc_layer(accp1, degp, W2, b2, True)
    (accp2,) = _sc_agg(h2, srcp, dstp)
    return _tc_layer(accp2, degp, W3, b3, False)[:_N]


# speedup vs baseline: 1.3998x; 1.0325x over previous
import functools

import jax
import jax.numpy as jnp
from jax import lax
from jax.experimental import pallas as pl
from jax.experimental.pallas import tpu as pltpu
from jax.experimental.pallas import tpu_sc as plsc

_N = 10000
_D = 128
_NC = 2
_NS = 16
_NW = _NC * _NS
_L = 16
_CH = 128
_NBUF = 2
_C0 = 136
_C1 = 24
_NCH = _NS * (_C0 + _C1)
_NPAD = 10240
_RPT = _NPAD // _NS


def _zero_1d(ref, n):
    z = jnp.zeros((_L,), jnp.float32)

    def bd(i, c):
        ref[pl.ds(i * _L, _L)] = z
        return c

    lax.fori_loop(0, n // _L, bd, 0)


def _zero_2d(ref, nrows, ncols):
    z = jnp.zeros((_L,), jnp.float32)
    cpr = ncols // _L

    def bd(i, c):
        ref[i // cpr, pl.ds((i % cpr) * _L, _L)] = z
        return c

    lax.fori_loop(0, nrows * cpr, bd, 0)


def _make_sc_agg(with_deg):
    mesh = plsc.VectorSubcoreMesh(core_axis_name="c", subcore_axis_name="s")
    qc = 8
    out_type = [jax.ShapeDtypeStruct((_NC, _NPAD, _D), jnp.float32)]
    scratch = [
        pltpu.VMEM((2, qc, _CH), jnp.int32),
        pltpu.VMEM((2, qc, _CH), jnp.int32),
        pltpu.VMEM((_NBUF, _CH, _D), jnp.float32),
        pltpu.VMEM_SHARED((_NPAD, _D), jnp.float32),
        pltpu.SemaphoreType.DMA,
        pltpu.SemaphoreType.DMA,
        pltpu.SemaphoreType.DMA,
    ]
    if with_deg:
        out_type.append(jax.ShapeDtypeStruct((_NW, _NPAD), jnp.float32))
        scratch.append(pltpu.VMEM((_NPAD,), jnp.float32))

    def body(table_hbm, srcp_hbm, dstp_hbm, acc_hbm, *rest):
        if with_deg:
            (deg_hbm, src_v, dst_v, rows_v, acc_sh, gsem, ssem, isem,
             hist_v) = rest
        else:
            src_v, dst_v, rows_v, acc_sh, gsem, ssem, isem = rest
        cid = lax.axis_index("c")
        sid = lax.axis_index("s")
        wid = sid * _NC + cid

        _zero_2d(rows_v.at[0], _CH, _D)
        for k in range(_RPT // _CH):
            pltpu.sync_copy(rows_v.at[0],
                            acc_sh.at[pl.ds(sid * _RPT + k * _CH, _CH)])
        if with_deg:
            _zero_1d(hist_v, _NPAD)
        plsc.subcore_barrier()

        ones = jnp.ones((_L,), jnp.float32)

        def wait_gather(b):
            pltpu.make_async_copy(table_hbm.at[src_v.at[0, 0]],
                                  rows_v.at[b], gsem).wait()

        def wait_scatter(b):
            pltpu.make_async_copy(rows_v.at[b], acc_sh.at[dst_v.at[0, 0]],
                                  ssem).wait()

        def run_core(nchk, base):
            nb = nchk // qc

            def idx_load(q, par):
                off = base + q * qc
                pltpu.async_copy(srcp_hbm.at[pl.ds(off, qc)],
                                 src_v.at[par], isem)
                pltpu.async_copy(dstp_hbm.at[pl.ds(off, qc)],
                                 dst_v.at[par], isem)

            def idx_wait():
                for _ in range(2):
                    pltpu.make_async_copy(srcp_hbm.at[pl.ds(0, qc)],
                                          src_v.at[0], isem).wait()

            def start_gather(x, b):
                par = (x // qc) % 2
                pltpu.async_copy(table_hbm.at[src_v.at[par, x % qc]],
                                 rows_v.at[b], gsem)

            def start_scatter(x, b):
                par = (x // qc) % 2
                pltpu.async_copy(rows_v.at[b],
                                 acc_sh.at[dst_v.at[par, x % qc]], ssem,
                                 add=True)

            def hist_x(x):
                if with_deg:
                    par = (x // qc) % 2
                    s = x % qc
                    for j in range(_CH // _L):
                        plsc.addupdate_scatter(
                            hist_v, [dst_v[par, s, pl.ds(j * _L, _L)]],
                            ones)

            idx_load(0, 0)
            idx_wait()
            start_gather(0, 0)

            def substep(x, b):
                q = x // qc
                slot = x % qc
                wait_gather(b)

                @pl.when(x > 0)
                def _ws():
                    wait_scatter(1 - b)

                @pl.when(jnp.logical_and(slot == 2, q + 1 < nb))
                def _pf():
                    idx_load(q + 1, (q + 1) % 2)

                @pl.when(x + 1 < nchk)
                def _g():
                    @pl.when(slot == qc - 1)
                    def _wi():
                        idx_wait()

                    start_gather(x + 1, 1 - b)

                start_scatter(x, b)
                hist_x(x)

            def pair(p, c):
                substep(2 * p, 0)
                substep(2 * p + 1, 1)
                return c

            lax.fori_loop(0, nchk // 2, pair, 0)
            wait_scatter(1)

        @pl.when(cid == 0)
        def _core0():
            run_core(_C0, sid * _C0)

        if _C1 > 0:
            @pl.when(cid == 1)
            def _core1():
                run_core(_C1, _NS * _C0 + sid * _C1)

        plsc.subcore_barrier()

        r0 = sid * _RPT
        pltpu.sync_copy(acc_sh.at[pl.ds(r0, _RPT)],
                        acc_hbm.at[cid].at[pl.ds(r0, _RPT)])
        if with_deg:
            pltpu.sync_copy(hist_v, deg_hbm.at[wid])

    return pl.kernel(
        body, out_type=tuple(out_type), mesh=mesh, scratch_types=scratch,
        compiler_params=pltpu.CompilerParams(needs_layout_passes=False))


_sc_agg_deg = _make_sc_agg(True)
_sc_agg = _make_sc_agg(False)


def _tc_layer(p, degp, w, b, relu):
    br = 2048

    def body(p_ref, degp_ref, w_ref, b_ref, o_ref):
        deg = jnp.maximum(jnp.sum(degp_ref[...], axis=0), 1.0)
        s = p_ref[0] + p_ref[1]
        hn = s * (1.0 / deg)[:, None]
        y = jnp.dot(hn, w_ref[...], preferred_element_type=jnp.float32)
        y = y + b_ref[...]
        o_ref[...] = jnp.maximum(y, 0.0) if relu else y

    return pl.pallas_call(
        body,
        grid=(_NPAD // br,),
        in_specs=[
            pl.BlockSpec((_NC, br, _D), lambda i: (0, i, 0)),
            pl.BlockSpec((_NW, br), lambda i: (0, i)),
            pl.BlockSpec((_D, _D), lambda i: (0, 0)),
            pl.BlockSpec((1, _D), lambda i: (0, 0)),
        ],
        out_specs=pl.BlockSpec((br, _D), lambda i: (i, 0)),
        out_shape=jax.ShapeDtypeStruct((_NPAD, _D), jnp.float32),
    )(p, degp, w, b.reshape(1, _D))


def kernel(feature, edge_index, W1, b1, W2, b2, W3, b3):
    del W1, b1
    e = edge_index.shape[1]
    e_pad = _NCH * _CH
    src = jnp.concatenate(
        [edge_index[0], jnp.zeros((e_pad - e,), jnp.int32)])
    dst = jnp.concatenate(
        [edge_index[1],
         _N + jnp.arange(e_pad - e, dtype=jnp.int32) % (_NPAD - _N)])
    srcp = src.reshape(_NCH, _CH)
    dstp = dst.reshape(_NCH, _CH)

    accp1, degp = _sc_agg_deg(feature, srcp, dstp)
    h2 = _tc_layer(accp1, degp, W2, b2, True)
    (accp2,) = _sc_agg(h2, srcp, dstp)
    return _tc_layer(accp2, degp, W3, b3, False)[:_N]


# speedup vs baseline: 1.5041x; 1.0745x over previous
import functools

import jax
import jax.numpy as jnp
from jax import lax
from jax.experimental import pallas as pl
from jax.experimental.pallas import tpu as pltpu
from jax.experimental.pallas import tpu_sc as plsc

_N = 10000
_D = 128
_NC = 2
_NS = 16
_NW = _NC * _NS
_L = 16
_CH = 128
_NBUF = 2
_C0 = 144
_C1 = 16
_NCH = _NS * (_C0 + _C1)
_NPAD = 10240
_RPT = _NPAD // _NS


def _zero_1d(ref, n):
    z = jnp.zeros((_L,), jnp.float32)

    def bd(i, c):
        ref[pl.ds(i * _L, _L)] = z
        return c

    lax.fori_loop(0, n // _L, bd, 0)


def _zero_2d(ref, nrows, ncols):
    z = jnp.zeros((_L,), jnp.float32)
    cpr = ncols // _L

    def bd(i, c):
        ref[i // cpr, pl.ds((i % cpr) * _L, _L)] = z
        return c

    lax.fori_loop(0, nrows * cpr, bd, 0)


def _make_sc_agg(with_deg):
    mesh = plsc.VectorSubcoreMesh(core_axis_name="c", subcore_axis_name="s")
    qc = 8
    out_type = [jax.ShapeDtypeStruct((_NC, _NPAD, _D), jnp.float32)]
    scratch = [
        pltpu.VMEM((2, qc, _CH), jnp.int32),
        pltpu.VMEM((2, qc, _CH), jnp.int32),
        pltpu.VMEM((_NBUF, _CH, _D), jnp.float32),
        pltpu.VMEM_SHARED((_NPAD, _D), jnp.float32),
        pltpu.SemaphoreType.DMA,
        pltpu.SemaphoreType.DMA,
        pltpu.SemaphoreType.DMA,
    ]
    if with_deg:
        out_type.append(jax.ShapeDtypeStruct((_NW, _NPAD), jnp.float32))
        scratch.append(pltpu.VMEM((_NPAD,), jnp.float32))

    def body(table_hbm, srcp_hbm, dstp_hbm, acc_hbm, *rest):
        if with_deg:
            (deg_hbm, src_v, dst_v, rows_v, acc_sh, gsem, ssem, isem,
             hist_v) = rest
        else:
            src_v, dst_v, rows_v, acc_sh, gsem, ssem, isem = rest
        cid = lax.axis_index("c")
        sid = lax.axis_index("s")
        wid = sid * _NC + cid

        _zero_2d(rows_v.at[0], _CH, _D)
        for k in range(_RPT // _CH):
            pltpu.sync_copy(rows_v.at[0],
                            acc_sh.at[pl.ds(sid * _RPT + k * _CH, _CH)])
        if with_deg:
            _zero_1d(hist_v, _NPAD)
        plsc.subcore_barrier()

        ones = jnp.ones((_L,), jnp.float32)

        def wait_gather(b):
            pltpu.make_async_copy(table_hbm.at[src_v.at[0, 0]],
                                  rows_v.at[b], gsem).wait()

        def wait_scatter(b):
            pltpu.make_async_copy(rows_v.at[b], acc_sh.at[dst_v.at[0, 0]],
                                  ssem).wait()

        def run_core(nchk, base):
            nb = nchk // qc

            def idx_load(q, par):
                off = base + q * qc
                pltpu.async_copy(srcp_hbm.at[pl.ds(off, qc)],
                                 src_v.at[par], isem)
                pltpu.async_copy(dstp_hbm.at[pl.ds(off, qc)],
                                 dst_v.at[par], isem)

            def idx_wait():
                for _ in range(2):
                    pltpu.make_async_copy(srcp_hbm.at[pl.ds(0, qc)],
                                          src_v.at[0], isem).wait()

            def start_gather(x, b):
                par = (x // qc) % 2
                pltpu.async_copy(table_hbm.at[src_v.at[par, x % qc]],
                                 rows_v.at[b], gsem)

            def start_scatter(x, b):
                par = (x // qc) % 2
                pltpu.async_copy(rows_v.at[b],
                                 acc_sh.at[dst_v.at[par, x % qc]], ssem,
                                 add=True)

            def hist_x(x):
                if with_deg:
                    par = (x // qc) % 2
                    s = x % qc
                    for j in range(_CH // _L):
                        plsc.addupdate_scatter(
                            hist_v, [dst_v[par, s, pl.ds(j * _L, _L)]],
                            ones)

            idx_load(0, 0)
            idx_wait()
            start_gather(0, 0)

            def substep(x, b):
                q = x // qc
                slot = x % qc
                wait_gather(b)

                @pl.when(x > 0)
                def _ws():
                    wait_scatter(1 - b)

                @pl.when(jnp.logical_and(slot == 2, q + 1 < nb))
                def _pf():
                    idx_load(q + 1, (q + 1) % 2)

                @pl.when(x + 1 < nchk)
                def _g():
                    @pl.when(slot == qc - 1)
                    def _wi():
                        idx_wait()

                    start_gather(x + 1, 1 - b)

                start_scatter(x, b)
                hist_x(x)

            def pair(p, c):
                substep(2 * p, 0)
                substep(2 * p + 1, 1)
                return c

            lax.fori_loop(0, nchk // 2, pair, 0)
            wait_scatter(1)

        @pl.when(cid == 0)
        def _core0():
            run_core(_C0, sid * _C0)

        if _C1 > 0:
            @pl.when(cid == 1)
            def _core1():
                run_core(_C1, _NS * _C0 + sid * _C1)

        plsc.subcore_barrier()

        r0 = sid * _RPT
        pltpu.sync_copy(acc_sh.at[pl.ds(r0, _RPT)],
                        acc_hbm.at[cid].at[pl.ds(r0, _RPT)])
        if with_deg:
            pltpu.sync_copy(hist_v, deg_hbm.at[wid])

    return pl.kernel(
        body, out_type=tuple(out_type), mesh=mesh, scratch_types=scratch,
        compiler_params=pltpu.CompilerParams(needs_layout_passes=False))


_sc_agg_deg = _make_sc_agg(True)
_sc_agg = _make_sc_agg(False)


def _tc_layer(p, degp, w, b, relu):
    br = 2048

    def body(p_ref, degp_ref, w_ref, b_ref, o_ref):
        deg = jnp.maximum(jnp.sum(degp_ref[...], axis=0), 1.0)
        s = p_ref[0] + p_ref[1]
        hn = s * (1.0 / deg)[:, None]
        y = jnp.dot(hn, w_ref[...], preferred_element_type=jnp.float32)
        y = y + b_ref[...]
        o_ref[...] = jnp.maximum(y, 0.0) if relu else y

    return pl.pallas_call(
        body,
        grid=(_NPAD // br,),
        in_specs=[
            pl.BlockSpec((_NC, br, _D), lambda i: (0, i, 0)),
            pl.BlockSpec((_NW, br), lambda i: (0, i)),
            pl.BlockSpec((_D, _D), lambda i: (0, 0)),
            pl.BlockSpec((1, _D), lambda i: (0, 0)),
        ],
        out_specs=pl.BlockSpec((br, _D), lambda i: (i, 0)),
        out_shape=jax.ShapeDtypeStruct((_NPAD, _D), jnp.float32),
    )(p, degp, w, b.reshape(1, _D))


def kernel(feature, edge_index, W1, b1, W2, b2, W3, b3):
    del W1, b1
    e = edge_index.shape[1]
    e_pad = _NCH * _CH
    src = jnp.concatenate(
        [edge_index[0], jnp.zeros((e_pad - e,), jnp.int32)])
    dst = jnp.concatenate(
        [edge_index[1],
         _N + jnp.arange(e_pad - e, dtype=jnp.int32) % (_NPAD - _N)])
    srcp = src.reshape(_NCH, _CH)
    dstp = dst.reshape(_NCH, _CH)

    accp1, degp = _sc_agg_deg(feature, srcp, dstp)
    h2 = _tc_layer(accp1, degp, W2, b2, True)
    (accp2,) = _sc_agg(h2, srcp, dstp)
    return _tc_layer(accp2, degp, W3, b3, False)[:_N]
